# initial kernel scaffold (unmeasured)
import jax
import jax.numpy as jnp
from jax import lax
from jax.experimental import pallas as pl
from jax.experimental.pallas import tpu as pltpu

N_DEV = 16


def kernel(A, B):
    m, k = A.shape
    _, n = B.shape
    chunk = m // N_DEV

    def body(a_ref, b_ref, out_ref, p_ref, comm_ref,
             send_sems, recv_sems, credit_sem):
        my = lax.axis_index("i")
        left = lax.rem(my + N_DEV - 1, N_DEV)
        right = lax.rem(my + 1, N_DEV)

        barrier_sem = pltpu.get_barrier_semaphore()
        for nbr in (left, right):
            pl.semaphore_signal(barrier_sem, inc=1, device_id=(nbr,),
                                device_id_type=pl.DeviceIdType.MESH)
        pl.semaphore_wait(barrier_sem, 2)

        p_ref[:, :] = jnp.dot(
            a_ref[:, :].astype(jnp.bfloat16),
            b_ref[:, :].astype(jnp.bfloat16),
            preferred_element_type=jnp.float32,
        )

        c0 = lax.rem(my + N_DEV - 1, N_DEV)
        comm_ref[0, :, :] = p_ref[pl.ds(c0 * chunk, chunk), :].astype(jnp.bfloat16)

        for h in range(N_DEV - 1):
            send_slot = h % 2
            recv_slot = (h + 1) % 2
            if h >= 1:
                pl.semaphore_wait(credit_sem, 1)
            rdma = pltpu.make_async_remote_copy(
                src_ref=comm_ref.at[send_slot],
                dst_ref=comm_ref.at[recv_slot],
                send_sem=send_sems.at[send_slot],
                recv_sem=recv_sems.at[recv_slot],
                device_id=(right,),
                device_id_type=pl.DeviceIdType.MESH,
            )
            rdma.start()
            rdma.wait_send()
            pl.semaphore_signal(credit_sem, inc=1, device_id=(left,),
                                device_id_type=pl.DeviceIdType.MESH)
            rdma.wait_recv()
            c = lax.rem(my + 2 * N_DEV - 2 - h, N_DEV)
            acc = (comm_ref[recv_slot, :, :].astype(jnp.float32)
                   + p_ref[pl.ds(c * chunk, chunk), :])
            if h < N_DEV - 2:
                comm_ref[recv_slot, :, :] = acc.astype(jnp.bfloat16)
            else:
                out_ref[:, :] = acc

    return pl.pallas_call(
        body,
        out_shape=jax.ShapeDtypeStruct((chunk, n), jnp.float32),
        in_specs=[
            pl.BlockSpec(memory_space=pltpu.VMEM),
            pl.BlockSpec(memory_space=pltpu.VMEM),
        ],
        out_specs=pl.BlockSpec(memory_space=pltpu.VMEM),
        scratch_shapes=[
            pltpu.VMEM((m, n), jnp.float32),
            pltpu.VMEM((2, chunk, n), jnp.bfloat16),
            pltpu.SemaphoreType.DMA((2,)),
            pltpu.SemaphoreType.DMA((2,)),
            pltpu.SemaphoreType.REGULAR,
        ],
        compiler_params=pltpu.CompilerParams(collective_id=0),
    )(A, B)


# baseline (device time: 149788 ns/iter reference)
import jax
import jax.numpy as jnp
from jax import lax
from jax.experimental import pallas as pl
from jax.experimental.pallas import tpu as pltpu

N_DEV = 16


def kernel(A, B):
    m, k = A.shape
    _, n = B.shape
    chunk = m // N_DEV

    def body(a_ref, b_ref, out_ref, p_ref, comm_ref,
             send_sems, recv_sems, credit_sem):
        my = lax.axis_index("i")
        left = lax.rem(my + N_DEV - 1, N_DEV)
        right = lax.rem(my + 1, N_DEV)

        barrier_sem = pltpu.get_barrier_semaphore()
        for nbr in (left, right):
            pl.semaphore_signal(barrier_sem, inc=1, device_id=(nbr,),
                                device_id_type=pl.DeviceIdType.MESH)
        pl.semaphore_wait(barrier_sem, 2)

        p_ref[:, :] = jnp.dot(
            a_ref[:, :].astype(jnp.bfloat16),
            b_ref[:, :].astype(jnp.bfloat16),
            preferred_element_type=jnp.float32,
        )

        c0 = lax.rem(my + N_DEV - 1, N_DEV)
        comm_ref[0, :, :] = p_ref[pl.ds(c0 * chunk, chunk), :].astype(jnp.bfloat16)

        for h in range(N_DEV - 1):
            send_slot = h % 2
            recv_slot = (h + 1) % 2
            if h >= 1:
                pl.semaphore_wait(credit_sem, 1)
            rdma = pltpu.make_async_remote_copy(
                src_ref=comm_ref.at[send_slot],
                dst_ref=comm_ref.at[recv_slot],
                send_sem=send_sems.at[send_slot],
                recv_sem=recv_sems.at[recv_slot],
                device_id=(right,),
                device_id_type=pl.DeviceIdType.MESH,
            )
            rdma.start()
            rdma.wait_send()
            if h < N_DEV - 2:
                pl.semaphore_signal(credit_sem, inc=1, device_id=(left,),
                                    device_id_type=pl.DeviceIdType.MESH)
            rdma.wait_recv()
            c = lax.rem(my + 2 * N_DEV - 2 - h, N_DEV)
            acc = (comm_ref[recv_slot, :, :].astype(jnp.float32)
                   + p_ref[pl.ds(c * chunk, chunk), :])
            if h < N_DEV - 2:
                comm_ref[recv_slot, :, :] = acc.astype(jnp.bfloat16)
            else:
                out_ref[:, :] = acc

    return pl.pallas_call(
        body,
        out_shape=jax.ShapeDtypeStruct((chunk, n), jnp.float32),
        in_specs=[
            pl.BlockSpec(memory_space=pltpu.VMEM),
            pl.BlockSpec(memory_space=pltpu.VMEM),
        ],
        out_specs=pl.BlockSpec(memory_space=pltpu.VMEM),
        scratch_shapes=[
            pltpu.VMEM((m, n), jnp.float32),
            pltpu.VMEM((2, chunk, n), jnp.bfloat16),
            pltpu.SemaphoreType.DMA((2,)),
            pltpu.SemaphoreType.DMA((2,)),
            pltpu.SemaphoreType.REGULAR,
        ],
        compiler_params=pltpu.CompilerParams(collective_id=0),
    )(A, B)


# device time: 96907 ns/iter; 1.5457x vs baseline; 1.5457x over previous
import jax
import jax.numpy as jnp
from jax import lax
from jax.experimental import pallas as pl
from jax.experimental.pallas import tpu as pltpu

N_DEV = 16

RING = [0, 1, 5, 9, 13, 14, 10, 6, 2, 3, 7, 11, 15, 12, 8, 4]


def kernel(A, B):
    m, k = A.shape
    _, n = B.shape
    chunk = m // N_DEV
    nh = n // 2

    my = lax.axis_index("i")
    r_arr = jnp.array(RING, jnp.int32)
    kpos = jnp.argmax(r_arr == my).astype(jnp.int32)
    succ = r_arr[(kpos + 1) % N_DEV]
    pred = r_arr[(kpos - 1) % N_DEV]
    h_idx = jnp.arange(N_DEV - 1, dtype=jnp.int32)
    cw_acc = r_arr[(kpos - 2 - h_idx) % N_DEV]
    ccw_acc = r_arr[(kpos + 2 + h_idx) % N_DEV]
    params = jnp.concatenate(
        [succ[None], pred[None], cw_acc, ccw_acc]).astype(jnp.int32)

    def body(a_ref, b_ref, prm_ref, out_ref, p_ref, cw_ref, ccw_ref,
             cw_send, cw_recv, ccw_send, ccw_recv, cw_credit, ccw_credit):
        succ = prm_ref[0]
        pred = prm_ref[1]

        barrier_sem = pltpu.get_barrier_semaphore()
        for nbr in (succ, pred):
            pl.semaphore_signal(barrier_sem, inc=1, device_id=(nbr,),
                                device_id_type=pl.DeviceIdType.MESH)
        pl.semaphore_wait(barrier_sem, 2)

        p_ref[:, :] = jnp.dot(
            a_ref[:, :].astype(jnp.bfloat16),
            b_ref[:, :].astype(jnp.bfloat16),
            preferred_element_type=jnp.float32,
        )

        cw_ref[0, :, :] = p_ref[pl.ds(pred * chunk, chunk), 0:nh].astype(jnp.bfloat16)
        ccw_ref[0, :, :] = p_ref[pl.ds(succ * chunk, chunk), nh:n].astype(jnp.bfloat16)

        for h in range(N_DEV - 1):
            ss = h % 2
            rs = (h + 1) % 2
            if h >= 1:
                pl.semaphore_wait(cw_credit, 1)
                pl.semaphore_wait(ccw_credit, 1)
            rd_cw = pltpu.make_async_remote_copy(
                src_ref=cw_ref.at[ss], dst_ref=cw_ref.at[rs],
                send_sem=cw_send.at[ss], recv_sem=cw_recv.at[rs],
                device_id=(succ,), device_id_type=pl.DeviceIdType.MESH,
            )
            rd_ccw = pltpu.make_async_remote_copy(
                src_ref=ccw_ref.at[ss], dst_ref=ccw_ref.at[rs],
                send_sem=ccw_send.at[ss], recv_sem=ccw_recv.at[rs],
                device_id=(pred,), device_id_type=pl.DeviceIdType.MESH,
            )
            rd_cw.start()
            rd_ccw.start()
            rd_cw.wait_send()
            rd_ccw.wait_send()
            if h < N_DEV - 2:
                pl.semaphore_signal(cw_credit, inc=1, device_id=(pred,),
                                    device_id_type=pl.DeviceIdType.MESH)
                pl.semaphore_signal(ccw_credit, inc=1, device_id=(succ,),
                                    device_id_type=pl.DeviceIdType.MESH)

            rd_cw.wait_recv()
            c1 = prm_ref[2 + h]
            acc1 = (cw_ref[rs, :, :].astype(jnp.float32)
                    + p_ref[pl.ds(c1 * chunk, chunk), 0:nh])
            if h < N_DEV - 2:
                cw_ref[rs, :, :] = acc1.astype(jnp.bfloat16)
            else:
                out_ref[:, 0:nh] = acc1

            rd_ccw.wait_recv()
            c2 = prm_ref[2 + (N_DEV - 1) + h]
            acc2 = (ccw_ref[rs, :, :].astype(jnp.float32)
                    + p_ref[pl.ds(c2 * chunk, chunk), nh:n])
            if h < N_DEV - 2:
                ccw_ref[rs, :, :] = acc2.astype(jnp.bfloat16)
            else:
                out_ref[:, nh:n] = acc2

    return pl.pallas_call(
        body,
        out_shape=jax.ShapeDtypeStruct((chunk, n), jnp.float32),
        in_specs=[
            pl.BlockSpec(memory_space=pltpu.VMEM),
            pl.BlockSpec(memory_space=pltpu.VMEM),
            pl.BlockSpec(memory_space=pltpu.SMEM),
        ],
        out_specs=pl.BlockSpec(memory_space=pltpu.VMEM),
        scratch_shapes=[
            pltpu.VMEM((m, n), jnp.float32),
            pltpu.VMEM((2, chunk, nh), jnp.bfloat16),
            pltpu.VMEM((2, chunk, nh), jnp.bfloat16),
            pltpu.SemaphoreType.DMA((2,)),
            pltpu.SemaphoreType.DMA((2,)),
            pltpu.SemaphoreType.DMA((2,)),
            pltpu.SemaphoreType.DMA((2,)),
            pltpu.SemaphoreType.REGULAR,
            pltpu.SemaphoreType.REGULAR,
        ],
        compiler_params=pltpu.CompilerParams(collective_id=0),
    )(A, B, params)


# device time: 82053 ns/iter; 1.8255x vs baseline; 1.1810x over previous
import jax
import jax.numpy as jnp
from jax import lax
from jax.experimental import pallas as pl
from jax.experimental.pallas import tpu as pltpu

N_DEV = 16
S = 2

RING = [0, 1, 5, 9, 13, 14, 10, 6, 2, 3, 7, 11, 15, 12, 8, 4]


def kernel(A, B):
    m, k = A.shape
    _, n = B.shape
    chunk = m // N_DEV
    nh = n // 2
    w = nh // S

    my = lax.axis_index("i")
    r_arr = jnp.array(RING, jnp.int32)
    kpos = jnp.argmax(r_arr == my).astype(jnp.int32)
    succ = r_arr[(kpos + 1) % N_DEV]
    pred = r_arr[(kpos - 1) % N_DEV]
    h_idx = jnp.arange(N_DEV - 1, dtype=jnp.int32)
    cw_acc = r_arr[(kpos - 2 - h_idx) % N_DEV]
    ccw_acc = r_arr[(kpos + 2 + h_idx) % N_DEV]
    params = jnp.concatenate(
        [succ[None], pred[None], cw_acc, ccw_acc]).astype(jnp.int32)

    n_hops = N_DEV - 1

    def body(a_ref, b_ref, prm_ref, out_ref, ab_ref, bb_ref, pc_cw, pc_ccw,
             *stream_refs):
        succ = prm_ref[0]
        pred = prm_ref[1]

        streams = []
        for j, (d, lo) in enumerate(
                [(d, j2 * w) for j2 in range(S) for d in (0, 1)]):
            buf, snd, rcv, cred = stream_refs[4 * j: 4 * j + 4]
            streams.append(dict(
                buf=buf, snd=snd, rcv=rcv, cred=cred, d=d, lo=lo,
                dst=succ if d == 0 else pred,
                ups=pred if d == 0 else succ,
                pc=pc_cw if d == 0 else pc_ccw,
                rdmas=[],
            ))

        barrier_sem = pltpu.get_barrier_semaphore()
        for nbr in (succ, pred):
            pl.semaphore_signal(barrier_sem, inc=1, device_id=(nbr,),
                                device_id_type=pl.DeviceIdType.MESH)
        pl.semaphore_wait(barrier_sem, 2)

        ab_ref[:, :] = a_ref[:, :].astype(jnp.bfloat16)
        bb_ref[:, :] = b_ref[:, :].astype(jnp.bfloat16)

        def chunk_mm(c, d, out):
            out[:, :] = jnp.dot(
                ab_ref[pl.ds(c * chunk, chunk), :],
                bb_ref[:, d * nh:(d + 1) * nh],
                preferred_element_type=jnp.float32,
            )

        def start_hop(s, h):
            rd = pltpu.make_async_remote_copy(
                src_ref=s['buf'].at[h % 2],
                dst_ref=s['buf'].at[(h + 1) % 2],
                send_sem=s['snd'].at[h % 2],
                recv_sem=s['rcv'].at[(h + 1) % 2],
                device_id=(s['dst'],),
                device_id_type=pl.DeviceIdType.MESH,
            )
            rd.start()
            s['rdmas'].append(rd)

        chunk_mm(pred, 0, pc_cw)
        chunk_mm(succ, 1, pc_ccw)
        for s in streams:
            s['buf'][0, :, :] = s['pc'][:, s['lo']:s['lo'] + w].astype(jnp.bfloat16)
        for s in streams:
            start_hop(s, 0)
        chunk_mm(prm_ref[2], 0, pc_cw)
        chunk_mm(prm_ref[2 + n_hops], 1, pc_ccw)

        for h in range(n_hops):
            rs = (h + 1) % 2
            last = h == n_hops - 1
            for s in streams:
                rd = s['rdmas'][h]
                rd.wait_recv()
                acc = (s['buf'][rs, :, :].astype(jnp.float32)
                       + s['pc'][:, s['lo']:s['lo'] + w])
                if last:
                    base = s['d'] * nh + s['lo']
                    out_ref[:, base:base + w] = acc
                else:
                    s['buf'][rs, :, :] = acc.astype(jnp.bfloat16)
                    rd.wait_send()
                    pl.semaphore_signal(s['cred'], inc=1,
                                        device_id=(s['ups'],),
                                        device_id_type=pl.DeviceIdType.MESH)
                    pl.semaphore_wait(s['cred'], 1)
                    start_hop(s, h + 1)
            if not last:
                chunk_mm(prm_ref[2 + h + 1], 0, pc_cw)
                chunk_mm(prm_ref[2 + n_hops + h + 1], 1, pc_ccw)

        for s in streams:
            s['rdmas'][n_hops - 1].wait_send()

    stream_scratch = []
    for _ in range(2 * S):
        stream_scratch += [
            pltpu.VMEM((2, chunk, w), jnp.bfloat16),
            pltpu.SemaphoreType.DMA((2,)),
            pltpu.SemaphoreType.DMA((2,)),
            pltpu.SemaphoreType.REGULAR,
        ]

    return pl.pallas_call(
        body,
        out_shape=jax.ShapeDtypeStruct((chunk, n), jnp.float32),
        in_specs=[
            pl.BlockSpec(memory_space=pltpu.VMEM),
            pl.BlockSpec(memory_space=pltpu.VMEM),
            pl.BlockSpec(memory_space=pltpu.SMEM),
        ],
        out_specs=pl.BlockSpec(memory_space=pltpu.VMEM),
        scratch_shapes=[
            pltpu.VMEM((m, k), jnp.bfloat16),
            pltpu.VMEM((k, n), jnp.bfloat16),
            pltpu.VMEM((chunk, nh), jnp.float32),
            pltpu.VMEM((chunk, nh), jnp.float32),
        ] + stream_scratch,
        compiler_params=pltpu.CompilerParams(collective_id=0),
    )(A, B, params)


# device time: 81845 ns/iter; 1.8301x vs baseline; 1.0025x over previous
import jax
import jax.numpy as jnp
from jax import lax
from jax.experimental import pallas as pl
from jax.experimental.pallas import tpu as pltpu

N_DEV = 16
S = 2

RING = [0, 1, 5, 9, 13, 14, 10, 6, 2, 3, 7, 11, 15, 12, 8, 4]


def kernel(A, B):
    m, k = A.shape
    _, n = B.shape
    chunk = m // N_DEV
    nh = n // 2
    w = nh // S

    my = lax.axis_index("i")
    r_arr = jnp.array(RING, jnp.int32)
    kpos = jnp.argmax(r_arr == my).astype(jnp.int32)
    succ = r_arr[(kpos + 1) % N_DEV]
    pred = r_arr[(kpos - 1) % N_DEV]
    h_idx = jnp.arange(N_DEV - 1, dtype=jnp.int32)
    cw_acc = r_arr[(kpos - 2 - h_idx) % N_DEV]
    ccw_acc = r_arr[(kpos + 2 + h_idx) % N_DEV]
    params = jnp.concatenate(
        [succ[None], pred[None], cw_acc, ccw_acc]).astype(jnp.int32)

    n_hops = N_DEV - 1

    def body(a_ref, b_ref, prm_ref, out_ref, ab_ref, bb_ref, pc_cw, pc_ccw,
             *stream_refs):
        succ = prm_ref[0]
        pred = prm_ref[1]

        streams = []
        for j, (d, lo) in enumerate(
                [(d, j2 * w) for j2 in range(S) for d in (0, 1)]):
            buf, snd, rcv, cred = stream_refs[4 * j: 4 * j + 4]
            streams.append(dict(
                buf=buf, snd=snd, rcv=rcv, cred=cred, d=d, lo=lo,
                dst=succ if d == 0 else pred,
                ups=pred if d == 0 else succ,
                pc=pc_cw if d == 0 else pc_ccw,
                rdmas=[],
            ))

        barrier_sem = pltpu.get_barrier_semaphore()
        for nbr in (succ, pred):
            pl.semaphore_signal(barrier_sem, inc=1, device_id=(nbr,),
                                device_id_type=pl.DeviceIdType.MESH)
        pl.semaphore_wait(barrier_sem, 2)

        ab_ref[:, :] = a_ref[:, :].astype(jnp.bfloat16)
        bb_ref[:, :] = b_ref[:, :].astype(jnp.bfloat16)

        def chunk_mm(c, d, out):
            out[:, :] = jnp.dot(
                ab_ref[pl.ds(c * chunk, chunk), :],
                bb_ref[:, d * nh:(d + 1) * nh],
                preferred_element_type=jnp.float32,
            ).astype(jnp.bfloat16)

        def start_hop(s, h):
            rd = pltpu.make_async_remote_copy(
                src_ref=s['buf'].at[h % 2],
                dst_ref=s['buf'].at[(h + 1) % 2],
                send_sem=s['snd'].at[h % 2],
                recv_sem=s['rcv'].at[(h + 1) % 2],
                device_id=(s['dst'],),
                device_id_type=pl.DeviceIdType.MESH,
            )
            rd.start()
            s['rdmas'].append(rd)

        chunk_mm(pred, 0, pc_cw)
        chunk_mm(succ, 1, pc_ccw)
        for s in streams:
            s['buf'][0, :, :] = s['pc'][:, s['lo']:s['lo'] + w]
        for s in streams:
            start_hop(s, 0)
        chunk_mm(prm_ref[2], 0, pc_cw)
        chunk_mm(prm_ref[2 + n_hops], 1, pc_ccw)

        for h in range(n_hops):
            rs = (h + 1) % 2
            last = h == n_hops - 1
            for s in streams:
                rd = s['rdmas'][h]
                rd.wait_recv()
                acc = s['buf'][rs, :, :] + s['pc'][:, s['lo']:s['lo'] + w]
                if last:
                    base = s['d'] * nh + s['lo']
                    out_ref[:, base:base + w] = acc.astype(jnp.float32)
                else:
                    s['buf'][rs, :, :] = acc
                    rd.wait_send()
                    pl.semaphore_signal(s['cred'], inc=1,
                                        device_id=(s['ups'],),
                                        device_id_type=pl.DeviceIdType.MESH)
                    pl.semaphore_wait(s['cred'], 1)
                    start_hop(s, h + 1)
            if not last:
                chunk_mm(prm_ref[2 + h + 1], 0, pc_cw)
                chunk_mm(prm_ref[2 + n_hops + h + 1], 1, pc_ccw)

        for s in streams:
            s['rdmas'][n_hops - 1].wait_send()

    stream_scratch = []
    for _ in range(2 * S):
        stream_scratch += [
            pltpu.VMEM((2, chunk, w), jnp.bfloat16),
            pltpu.SemaphoreType.DMA((2,)),
            pltpu.SemaphoreType.DMA((2,)),
            pltpu.SemaphoreType.REGULAR,
        ]

    return pl.pallas_call(
        body,
        out_shape=jax.ShapeDtypeStruct((chunk, n), jnp.float32),
        in_specs=[
            pl.BlockSpec(memory_space=pltpu.VMEM),
            pl.BlockSpec(memory_space=pltpu.VMEM),
            pl.BlockSpec(memory_space=pltpu.SMEM),
        ],
        out_specs=pl.BlockSpec(memory_space=pltpu.VMEM),
        scratch_shapes=[
            pltpu.VMEM((m, k), jnp.bfloat16),
            pltpu.VMEM((k, n), jnp.bfloat16),
            pltpu.VMEM((chunk, nh), jnp.bfloat16),
            pltpu.VMEM((chunk, nh), jnp.bfloat16),
        ] + stream_scratch,
        compiler_params=pltpu.CompilerParams(collective_id=0),
    )(A, B, params)
